# baseline (speedup 1.0000x reference)
import jax
import jax.numpy as jnp
from jax import lax
from jax.experimental import pallas as pl
from jax.experimental.pallas import tpu as pltpu
from jax.experimental.pallas import tpu_sc as plsc

N = 10000
NPAD = 10240
E = 320000
D = 128
B = 64
NW = 32
CH = 128
NCHUNK = 80
NCHUNK2 = 160
EPAD = NW * NCHUNK * CH
HD = D // 2
ROWS_PER_SUB = NPAD // 16

_HIGH = jax.lax.Precision.HIGHEST


def _vector_mesh():
    return plsc.VectorSubcoreMesh(core_axis_name="c", subcore_axis_name="s")


_SC_PARAMS = pltpu.CompilerParams(use_tc_tiling_on_sc=False)
_SC_PARAMS_NL = pltpu.CompilerParams(use_tc_tiling_on_sc=False,
                                     needs_layout_passes=False)


CHD = 80
NCHD = 125


def _deg_body(dst_hbm, out_hbm, dstv, hist):
    cid = lax.axis_index("c")
    sid = lax.axis_index("s")
    wid = cid * 16 + sid

    zero = jnp.zeros((16,), jnp.float32)
    ones = jnp.ones((16,), jnp.float32)

    @pl.loop(0, D // 16)
    def _(k):
        @pl.loop(0, CHD)
        def _(i):
            hist[i, pl.ds(k * 16, 16)] = zero

    pltpu.sync_copy(dst_hbm.at[wid], dstv)

    @pl.loop(0, NCHD)
    def _(j):
        for k in range(CHD // 16):
            v = dstv[j, pl.ds(k * 16, 16)]
            plsc.addupdate_scatter(hist, [v >> 7, v & 127], ones)

    pltpu.sync_copy(hist, out_hbm.at[wid])


def _deg_call(dst_deg):
    k = pl.kernel(
        _deg_body,
        out_type=jax.ShapeDtypeStruct((NW, CHD, D), jnp.float32),
        mesh=_vector_mesh(),
        compiler_params=_SC_PARAMS_NL,
        scratch_types=[
            pltpu.VMEM((NCHD, CHD), jnp.int32),
            pltpu.VMEM((CHD, D), jnp.float32),
        ],
    )
    return k(dst_deg)


IH = NCHUNK // 2
SPLIT = 4
VB = 2 * SPLIT
VW = CH // SPLIT


def _agg_body(g_hbm, src_hbm, dst_hbm, out_hbm, srcv, dstv, b0, b1, zbuf,
              acc, *sems):
    cid = lax.axis_index("c")
    sid = lax.axis_index("s")
    wid = cid * 16 + sid
    bufs = (b0, b1)
    gsem = sems[:VB]
    ssem = sems[VB:]

    @pl.when(cid == 0)
    def _():
        pltpu.sync_copy(
            g_hbm.at[pl.ds(sid * ROWS_PER_SUB, ROWS_PER_SUB)],
            acc.at[pl.ds(sid * ROWS_PER_SUB, ROWS_PER_SUB)],
        )

    @pl.when(cid == 1)
    def _():
        lanes = lax.iota(jnp.int32, 16)
        zero = jnp.where(lanes == 0, 0.0, 0.0).astype(jnp.float32)

        @pl.loop(0, 16)
        def _(i):
            for k in range(D // 16):
                zbuf[i, pl.ds(k * 16, 16)] = zero

        @pl.loop(0, ROWS_PER_SUB // 16)
        def _(r):
            pltpu.sync_copy(
                zbuf, acc.at[pl.ds(sid * ROWS_PER_SUB + r * 16, 16)])

    def vbuf(k):
        return bufs[k // SPLIT].at[pl.ds((k % SPLIT) * VW, VW)]

    def vsrc(j, k):
        return srcv.at[j, pl.ds((k % SPLIT) * VW, VW)]

    def vdst(j, k):
        return dstv.at[j, pl.ds((k % SPLIT) * VW, VW)]

    for half in range(2):
        pltpu.sync_copy(src_hbm.at[wid].at[pl.ds(half * IH, IH)], srcv)
        pltpu.sync_copy(dst_hbm.at[wid].at[pl.ds(half * IH, IH)], dstv)
        if half == 0:
            plsc.subcore_barrier()

        for k in range(VB):
            pltpu.async_copy(g_hbm.at[vsrc(k // SPLIT, k)], vbuf(k), gsem[k])

        @pl.loop(0, IH // 2 - 1)
        def _(t):
            for k in range(VB):
                j = t * 2 + k // SPLIT
                pltpu.make_async_copy(g_hbm.at[vsrc(j, k)], vbuf(k),
                                      gsem[k]).wait()
                pltpu.async_copy(vbuf(k), acc.at[vdst(j, k)], ssem[k],
                                 add=True)
            for k in range(VB):
                j = t * 2 + k // SPLIT
                pltpu.make_async_copy(vbuf(k), acc.at[vdst(j, k)],
                                      ssem[k]).wait()
                pltpu.async_copy(g_hbm.at[vsrc(j + 2, k)], vbuf(k), gsem[k])

        jl = IH - 2
        for k in range(VB):
            j = jl + k // SPLIT
            pltpu.make_async_copy(g_hbm.at[vsrc(j, k)], vbuf(k),
                                  gsem[k]).wait()
            pltpu.async_copy(vbuf(k), acc.at[vdst(j, k)], ssem[k], add=True)
        for k in range(VB):
            j = jl + k // SPLIT
            pltpu.make_async_copy(vbuf(k), acc.at[vdst(j, k)],
                                  ssem[k]).wait()

    plsc.subcore_barrier()
    pltpu.sync_copy(
        acc.at[pl.ds(sid * ROWS_PER_SUB, ROWS_PER_SUB)],
        out_hbm.at[cid].at[pl.ds(sid * ROWS_PER_SUB, ROWS_PER_SUB)],
    )


def _agg_call(g, src_p, dst_p):
    k = pl.kernel(
        _agg_body,
        out_type=jax.ShapeDtypeStruct((2, NPAD, D), jnp.float32),
        mesh=_vector_mesh(),
        scratch_types=[
            pltpu.VMEM((IH, CH), jnp.int32),
            pltpu.VMEM((IH, CH), jnp.int32),
            pltpu.VMEM((CH, D), jnp.float32),
            pltpu.VMEM((CH, D), jnp.float32),
            pltpu.VMEM((16, D), jnp.float32),
            pltpu.VMEM_SHARED((NPAD, D), jnp.float32),
        ] + [pltpu.SemaphoreType.DMA] * (2 * VB),
    )
    return k(g, src_p, dst_p)


def _tc0_body(xp_ref, w1_ref, h_ref):
    h_ref[...] = jnp.dot(xp_ref[...], w1_ref[...], precision=_HIGH,
                         preferred_element_type=jnp.float32)


def _tcdeg_body(degp_ref, dinv_ref):
    deg = jnp.sum(degp_ref[...], axis=0) + 1.0
    dinv_ref[...] = lax.rsqrt(deg)


def _tc1_body(h_ref, dinv_ref, g1_ref):
    g1_ref[...] = h_ref[...] * dinv_ref[...]


def _tc2_body(dinv_ref, p_ref, b_ref, w_ref, out_ref):
    dinv = dinv_ref[...]
    u = p_ref[0] + p_ref[1]
    z = jnp.maximum(u * dinv + b_ref[...], 0.0)
    out_ref[...] = jnp.dot(z, w_ref[...], precision=_HIGH,
                           preferred_element_type=jnp.float32) * dinv


def _tc3_body(dinv_ref, p_ref, b_ref, batch_ref, wl_ref, bl_ref,
              out_ref):
    dinv = dinv_ref[...]
    u = p_ref[0] + p_ref[1]
    z = jnp.maximum(u * dinv + b_ref[...], 0.0)
    gid = lax.broadcasted_iota(jnp.int32, (1, B), 1)
    m = (batch_ref[...] == gid).astype(jnp.float32)
    sums = lax.dot_general(m, z, (((0,), (0,)), ((), ())), precision=_HIGH,
                           preferred_element_type=jnp.float32)
    counts = jnp.sum(m, axis=0)[:, None]
    pooled = sums / jnp.maximum(counts, 1.0)
    out_ref[...] = jnp.dot(pooled, wl_ref[...], precision=_HIGH,
                           preferred_element_type=jnp.float32) + bl_ref[...]


def _tc_call(body, out_shape, *args):
    return pl.pallas_call(body, out_shape=out_shape)(*args)


def kernel(x, ei, batch, W1, b1, W2, b2, Wl, bl):
    pad = EPAD - E
    idx = jnp.arange(pad, dtype=jnp.int32)
    src_flat = jnp.concatenate([ei[0], idx])
    dst_flat = jnp.concatenate([ei[1], N + (idx & 127)])
    src_p = src_flat.reshape(NW, NCHUNK, CH)
    dst_p = dst_flat.reshape(NW, NCHUNK, CH)
    dst_deg = ei[1].reshape(NW, NCHD, CHD)
    x_p = jnp.pad(x, ((0, NPAD - N), (0, 0)))
    batch_p = jnp.pad(batch, (0, NPAD - N), constant_values=B).reshape(
        NPAD, 1)

    degp = _deg_call(dst_deg)
    dinv = _tc_call(_tcdeg_body,
                    jax.ShapeDtypeStruct((CHD, D), jnp.float32),
                    degp).reshape(NPAD, 1)

    h1 = _tc_call(_tc0_body, jax.ShapeDtypeStruct((NPAD, D), jnp.float32),
                  x_p, W1)
    g1 = _tc_call(_tc1_body, jax.ShapeDtypeStruct((NPAD, D), jnp.float32),
                  h1, dinv)
    p1 = _agg_call(g1, src_p, dst_p)
    g2 = _tc_call(_tc2_body, jax.ShapeDtypeStruct((NPAD, D), jnp.float32),
                  dinv, p1, b1.reshape(1, D), W2)
    p2 = _agg_call(g2, src_p, dst_p)
    out = _tc_call(_tc3_body, jax.ShapeDtypeStruct((B, D), jnp.float32),
                   dinv, p2, b2.reshape(1, D), batch_p, Wl,
                   bl.reshape(1, D))
    return out


# speedup vs baseline: 1.0252x; 1.0252x over previous
import jax
import jax.numpy as jnp
from jax import lax
from jax.experimental import pallas as pl
from jax.experimental.pallas import tpu as pltpu
from jax.experimental.pallas import tpu_sc as plsc

N = 10000
NPAD = 10240
E = 320000
D = 128
B = 64
NW = 32
CH = 128
NCHUNK = 80
NCHUNK2 = 160
EPAD = NW * NCHUNK * CH
HD = D // 2
ROWS_PER_SUB = NPAD // 16

_HIGH = jax.lax.Precision.DEFAULT


def _vector_mesh():
    return plsc.VectorSubcoreMesh(core_axis_name="c", subcore_axis_name="s")


_SC_PARAMS = pltpu.CompilerParams(use_tc_tiling_on_sc=False)
_SC_PARAMS_NL = pltpu.CompilerParams(use_tc_tiling_on_sc=False,
                                     needs_layout_passes=False)


CHD = 80
NCHD = 125


def _deg_body(dst_hbm, out_hbm, dstv, hist):
    cid = lax.axis_index("c")
    sid = lax.axis_index("s")
    wid = cid * 16 + sid

    zero = jnp.zeros((16,), jnp.float32)
    ones = jnp.ones((16,), jnp.float32)

    @pl.loop(0, D // 16)
    def _(k):
        @pl.loop(0, CHD)
        def _(i):
            hist[i, pl.ds(k * 16, 16)] = zero

    pltpu.sync_copy(dst_hbm.at[wid], dstv)

    @pl.loop(0, NCHD)
    def _(j):
        for k in range(CHD // 16):
            v = dstv[j, pl.ds(k * 16, 16)]
            plsc.addupdate_scatter(hist, [v >> 7, v & 127], ones)

    pltpu.sync_copy(hist, out_hbm.at[wid])


def _deg_call(dst_deg):
    k = pl.kernel(
        _deg_body,
        out_type=jax.ShapeDtypeStruct((NW, CHD, D), jnp.float32),
        mesh=_vector_mesh(),
        compiler_params=_SC_PARAMS_NL,
        scratch_types=[
            pltpu.VMEM((NCHD, CHD), jnp.int32),
            pltpu.VMEM((CHD, D), jnp.float32),
        ],
    )
    return k(dst_deg)


IH = NCHUNK // 2
SPLIT = 4
VB = 2 * SPLIT
VW = CH // SPLIT


def _agg_body(g_hbm, src_hbm, dst_hbm, out_hbm, srcv, dstv, b0, b1, zbuf,
              acc, *sems):
    cid = lax.axis_index("c")
    sid = lax.axis_index("s")
    wid = cid * 16 + sid
    bufs = (b0, b1)
    gsem = sems[:VB]
    ssem = sems[VB:]

    @pl.when(cid == 0)
    def _():
        pltpu.sync_copy(
            g_hbm.at[pl.ds(sid * ROWS_PER_SUB, ROWS_PER_SUB)],
            acc.at[pl.ds(sid * ROWS_PER_SUB, ROWS_PER_SUB)],
        )

    @pl.when(cid == 1)
    def _():
        lanes = lax.iota(jnp.int32, 16)
        zero = jnp.where(lanes == 0, 0.0, 0.0).astype(jnp.float32)

        @pl.loop(0, 16)
        def _(i):
            for k in range(D // 16):
                zbuf[i, pl.ds(k * 16, 16)] = zero

        @pl.loop(0, ROWS_PER_SUB // 16)
        def _(r):
            pltpu.sync_copy(
                zbuf, acc.at[pl.ds(sid * ROWS_PER_SUB + r * 16, 16)])

    def vbuf(k):
        return bufs[k // SPLIT].at[pl.ds((k % SPLIT) * VW, VW)]

    def vsrc(j, k):
        return srcv.at[j, pl.ds((k % SPLIT) * VW, VW)]

    def vdst(j, k):
        return dstv.at[j, pl.ds((k % SPLIT) * VW, VW)]

    for half in range(2):
        pltpu.sync_copy(src_hbm.at[wid].at[pl.ds(half * IH, IH)], srcv)
        pltpu.sync_copy(dst_hbm.at[wid].at[pl.ds(half * IH, IH)], dstv)
        if half == 0:
            plsc.subcore_barrier()

        for k in range(VB):
            pltpu.async_copy(g_hbm.at[vsrc(k // SPLIT, k)], vbuf(k), gsem[k])

        @pl.loop(0, IH // 2 - 1)
        def _(t):
            for k in range(VB):
                j = t * 2 + k // SPLIT
                pltpu.make_async_copy(g_hbm.at[vsrc(j, k)], vbuf(k),
                                      gsem[k]).wait()
                pltpu.async_copy(vbuf(k), acc.at[vdst(j, k)], ssem[k],
                                 add=True)
            for k in range(VB):
                j = t * 2 + k // SPLIT
                pltpu.make_async_copy(vbuf(k), acc.at[vdst(j, k)],
                                      ssem[k]).wait()
                pltpu.async_copy(g_hbm.at[vsrc(j + 2, k)], vbuf(k), gsem[k])

        jl = IH - 2
        for k in range(VB):
            j = jl + k // SPLIT
            pltpu.make_async_copy(g_hbm.at[vsrc(j, k)], vbuf(k),
                                  gsem[k]).wait()
            pltpu.async_copy(vbuf(k), acc.at[vdst(j, k)], ssem[k], add=True)
        for k in range(VB):
            j = jl + k // SPLIT
            pltpu.make_async_copy(vbuf(k), acc.at[vdst(j, k)],
                                  ssem[k]).wait()

    plsc.subcore_barrier()
    pltpu.sync_copy(
        acc.at[pl.ds(sid * ROWS_PER_SUB, ROWS_PER_SUB)],
        out_hbm.at[cid].at[pl.ds(sid * ROWS_PER_SUB, ROWS_PER_SUB)],
    )


def _agg_call(g, src_p, dst_p):
    k = pl.kernel(
        _agg_body,
        out_type=jax.ShapeDtypeStruct((2, NPAD, D), jnp.float32),
        mesh=_vector_mesh(),
        scratch_types=[
            pltpu.VMEM((IH, CH), jnp.int32),
            pltpu.VMEM((IH, CH), jnp.int32),
            pltpu.VMEM((CH, D), jnp.float32),
            pltpu.VMEM((CH, D), jnp.float32),
            pltpu.VMEM((16, D), jnp.float32),
            pltpu.VMEM_SHARED((NPAD, D), jnp.float32),
        ] + [pltpu.SemaphoreType.DMA] * (2 * VB),
    )
    return k(g, src_p, dst_p)


def _tc0_body(xp_ref, w1_ref, h_ref):
    h_ref[...] = jnp.dot(xp_ref[...], w1_ref[...], precision=_HIGH,
                         preferred_element_type=jnp.float32)


def _tcdeg_body(degp_ref, dinv_ref):
    deg = jnp.sum(degp_ref[...], axis=0) + 1.0
    dinv_ref[...] = lax.rsqrt(deg)


def _tc1_body(h_ref, dinv_ref, g1_ref):
    g1_ref[...] = h_ref[...] * dinv_ref[...]


def _tc2_body(dinv_ref, p_ref, b_ref, w_ref, out_ref):
    dinv = dinv_ref[...]
    u = p_ref[0] + p_ref[1]
    z = jnp.maximum(u * dinv + b_ref[...], 0.0)
    out_ref[...] = jnp.dot(z, w_ref[...], precision=_HIGH,
                           preferred_element_type=jnp.float32) * dinv


def _tc3_body(dinv_ref, p_ref, b_ref, batch_ref, wl_ref, bl_ref,
              out_ref):
    dinv = dinv_ref[...]
    u = p_ref[0] + p_ref[1]
    z = jnp.maximum(u * dinv + b_ref[...], 0.0)
    gid = lax.broadcasted_iota(jnp.int32, (1, B), 1)
    m = (batch_ref[...] == gid).astype(jnp.float32)
    sums = lax.dot_general(m, z, (((0,), (0,)), ((), ())), precision=_HIGH,
                           preferred_element_type=jnp.float32)
    counts = jnp.sum(m, axis=0)[:, None]
    pooled = sums / jnp.maximum(counts, 1.0)
    out_ref[...] = jnp.dot(pooled, wl_ref[...], precision=_HIGH,
                           preferred_element_type=jnp.float32) + bl_ref[...]


def _tc_call(body, out_shape, *args):
    return pl.pallas_call(body, out_shape=out_shape)(*args)


def kernel(x, ei, batch, W1, b1, W2, b2, Wl, bl):
    pad = EPAD - E
    idx = jnp.arange(pad, dtype=jnp.int32)
    src_flat = jnp.concatenate([ei[0], idx])
    dst_flat = jnp.concatenate([ei[1], N + (idx & 127)])
    src_p = src_flat.reshape(NW, NCHUNK, CH)
    dst_p = dst_flat.reshape(NW, NCHUNK, CH)
    dst_deg = ei[1].reshape(NW, NCHD, CHD)
    x_p = jnp.pad(x, ((0, NPAD - N), (0, 0)))
    batch_p = jnp.pad(batch, (0, NPAD - N), constant_values=B).reshape(
        NPAD, 1)

    degp = _deg_call(dst_deg)
    dinv = _tc_call(_tcdeg_body,
                    jax.ShapeDtypeStruct((CHD, D), jnp.float32),
                    degp).reshape(NPAD, 1)

    h1 = _tc_call(_tc0_body, jax.ShapeDtypeStruct((NPAD, D), jnp.float32),
                  x_p, W1)
    g1 = _tc_call(_tc1_body, jax.ShapeDtypeStruct((NPAD, D), jnp.float32),
                  h1, dinv)
    p1 = _agg_call(g1, src_p, dst_p)
    g2 = _tc_call(_tc2_body, jax.ShapeDtypeStruct((NPAD, D), jnp.float32),
                  dinv, p1, b1.reshape(1, D), W2)
    p2 = _agg_call(g2, src_p, dst_p)
    out = _tc_call(_tc3_body, jax.ShapeDtypeStruct((B, D), jnp.float32),
                   dinv, p2, b2.reshape(1, D), batch_p, Wl,
                   bl.reshape(1, D))
    return out


# speedup vs baseline: 1.0496x; 1.0237x over previous
import jax
import jax.numpy as jnp
from jax import lax
from jax.experimental import pallas as pl
from jax.experimental.pallas import tpu as pltpu
from jax.experimental.pallas import tpu_sc as plsc

N = 10000
NPAD = 10240
E = 320000
D = 128
B = 64
NW = 32
CH = 128
NCHUNK = 80
NCHUNK2 = 160
EPAD = NW * NCHUNK * CH
HD = D // 2
ROWS_PER_SUB = NPAD // 16

_HIGH = jax.lax.Precision.DEFAULT


def _vector_mesh():
    return plsc.VectorSubcoreMesh(core_axis_name="c", subcore_axis_name="s")


_SC_PARAMS = pltpu.CompilerParams(use_tc_tiling_on_sc=False)
_SC_PARAMS_NL = pltpu.CompilerParams(use_tc_tiling_on_sc=False,
                                     needs_layout_passes=False)


CHD = 80
NCHD = 125


def _deg_body(dst_hbm, out_hbm, dstv, hist):
    cid = lax.axis_index("c")
    sid = lax.axis_index("s")
    wid = cid * 16 + sid

    zero = jnp.zeros((16,), jnp.float32)
    ones = jnp.ones((16,), jnp.float32)

    @pl.loop(0, D // 16)
    def _(k):
        @pl.loop(0, CHD)
        def _(i):
            hist[i, pl.ds(k * 16, 16)] = zero

    pltpu.sync_copy(dst_hbm.at[wid], dstv)

    @pl.loop(0, NCHD)
    def _(j):
        for k in range(CHD // 16):
            v = dstv[j, pl.ds(k * 16, 16)]
            plsc.addupdate_scatter(hist, [v >> 7, v & 127], ones)

    pltpu.sync_copy(hist, out_hbm.at[wid])


def _deg_call(dst_deg):
    k = pl.kernel(
        _deg_body,
        out_type=jax.ShapeDtypeStruct((NW, CHD, D), jnp.float32),
        mesh=_vector_mesh(),
        compiler_params=_SC_PARAMS_NL,
        scratch_types=[
            pltpu.VMEM((NCHD, CHD), jnp.int32),
            pltpu.VMEM((CHD, D), jnp.float32),
        ],
    )
    return k(dst_deg)


IH = NCHUNK // 2
SPLIT = 4
VB = 2 * SPLIT
VW = CH // SPLIT


def _agg_body(g_hbm, src_hbm, dst_hbm, out_hbm, srcv, dstv, b0, b1, zbuf,
              acc, *sems):
    cid = lax.axis_index("c")
    sid = lax.axis_index("s")
    wid = cid * 16 + sid
    bufs = (b0, b1)
    gsem = sems[:VB]
    ssem = sems[VB:2 * VB]
    isem = sems[2 * VB]

    @pl.when(cid == 0)
    def _():
        pltpu.async_copy(
            g_hbm.at[pl.ds(sid * ROWS_PER_SUB, ROWS_PER_SUB)],
            acc.at[pl.ds(sid * ROWS_PER_SUB, ROWS_PER_SUB)], isem)

    @pl.when(cid == 1)
    def _():
        lanes = lax.iota(jnp.int32, 16)
        zero = jnp.where(lanes == 0, 0.0, 0.0).astype(jnp.float32)

        @pl.loop(0, 16)
        def _(i):
            for k in range(D // 16):
                zbuf[i, pl.ds(k * 16, 16)] = zero

        @pl.loop(0, ROWS_PER_SUB // 16)
        def _(r):
            pltpu.async_copy(
                zbuf, acc.at[pl.ds(sid * ROWS_PER_SUB + r * 16, 16)], isem)

    def _drain_init():
        @pl.when(cid == 0)
        def _():
            pltpu.make_async_copy(
                g_hbm.at[pl.ds(sid * ROWS_PER_SUB, ROWS_PER_SUB)],
                acc.at[pl.ds(sid * ROWS_PER_SUB, ROWS_PER_SUB)], isem).wait()

        @pl.when(cid == 1)
        def _():
            @pl.loop(0, ROWS_PER_SUB // 16)
            def _(r):
                pltpu.make_async_copy(
                    zbuf, acc.at[pl.ds(sid * ROWS_PER_SUB, 16)], isem).wait()

    def vbuf(k):
        return bufs[k // SPLIT].at[pl.ds((k % SPLIT) * VW, VW)]

    def vsrc(j, k):
        return srcv.at[j, pl.ds((k % SPLIT) * VW, VW)]

    def vdst(j, k):
        return dstv.at[j, pl.ds((k % SPLIT) * VW, VW)]

    for half in range(2):
        pltpu.sync_copy(src_hbm.at[wid].at[pl.ds(half * IH, IH)], srcv)
        pltpu.sync_copy(dst_hbm.at[wid].at[pl.ds(half * IH, IH)], dstv)

        for k in range(VB):
            pltpu.async_copy(g_hbm.at[vsrc(k // SPLIT, k)], vbuf(k), gsem[k])
        if half == 0:
            _drain_init()
            plsc.subcore_barrier()

        @pl.loop(0, IH // 2 - 1)
        def _(t):
            for k in range(VB):
                j = t * 2 + k // SPLIT
                pltpu.make_async_copy(g_hbm.at[vsrc(j, k)], vbuf(k),
                                      gsem[k]).wait()
                pltpu.async_copy(vbuf(k), acc.at[vdst(j, k)], ssem[k],
                                 add=True)
            for k in range(VB):
                j = t * 2 + k // SPLIT
                pltpu.make_async_copy(vbuf(k), acc.at[vdst(j, k)],
                                      ssem[k]).wait()
                pltpu.async_copy(g_hbm.at[vsrc(j + 2, k)], vbuf(k), gsem[k])

        jl = IH - 2
        for k in range(VB):
            j = jl + k // SPLIT
            pltpu.make_async_copy(g_hbm.at[vsrc(j, k)], vbuf(k),
                                  gsem[k]).wait()
            pltpu.async_copy(vbuf(k), acc.at[vdst(j, k)], ssem[k], add=True)
        for k in range(VB):
            j = jl + k // SPLIT
            pltpu.make_async_copy(vbuf(k), acc.at[vdst(j, k)],
                                  ssem[k]).wait()

    plsc.subcore_barrier()
    pltpu.sync_copy(
        acc.at[pl.ds(sid * ROWS_PER_SUB, ROWS_PER_SUB)],
        out_hbm.at[cid].at[pl.ds(sid * ROWS_PER_SUB, ROWS_PER_SUB)],
    )


def _agg_call(g, src_p, dst_p):
    k = pl.kernel(
        _agg_body,
        out_type=jax.ShapeDtypeStruct((2, NPAD, D), jnp.float32),
        mesh=_vector_mesh(),
        scratch_types=[
            pltpu.VMEM((IH, CH), jnp.int32),
            pltpu.VMEM((IH, CH), jnp.int32),
            pltpu.VMEM((CH, D), jnp.float32),
            pltpu.VMEM((CH, D), jnp.float32),
            pltpu.VMEM((16, D), jnp.float32),
            pltpu.VMEM_SHARED((NPAD, D), jnp.float32),
        ] + [pltpu.SemaphoreType.DMA] * (2 * VB + 1),
    )
    return k(g, src_p, dst_p)


def _tc0_body(xp_ref, w1_ref, h_ref):
    h_ref[...] = jnp.dot(xp_ref[...], w1_ref[...], precision=_HIGH,
                         preferred_element_type=jnp.float32)


def _tcdeg_body(degp_ref, dinv_ref):
    deg = jnp.sum(degp_ref[...], axis=0) + 1.0
    dinv_ref[...] = lax.rsqrt(deg)


def _tc1_body(h_ref, dinv_ref, g1_ref):
    g1_ref[...] = h_ref[...] * dinv_ref[...]


def _tc2_body(dinv_ref, p_ref, b_ref, w_ref, out_ref):
    dinv = dinv_ref[...]
    u = p_ref[0] + p_ref[1]
    z = jnp.maximum(u * dinv + b_ref[...], 0.0)
    out_ref[...] = jnp.dot(z, w_ref[...], precision=_HIGH,
                           preferred_element_type=jnp.float32) * dinv


def _tc3_body(dinv_ref, p_ref, b_ref, batch_ref, wl_ref, bl_ref,
              out_ref):
    dinv = dinv_ref[...]
    u = p_ref[0] + p_ref[1]
    z = jnp.maximum(u * dinv + b_ref[...], 0.0)
    gid = lax.broadcasted_iota(jnp.int32, (1, B), 1)
    m = (batch_ref[...] == gid).astype(jnp.float32)
    sums = lax.dot_general(m, z, (((0,), (0,)), ((), ())), precision=_HIGH,
                           preferred_element_type=jnp.float32)
    counts = jnp.sum(m, axis=0)[:, None]
    pooled = sums / jnp.maximum(counts, 1.0)
    out_ref[...] = jnp.dot(pooled, wl_ref[...], precision=_HIGH,
                           preferred_element_type=jnp.float32) + bl_ref[...]


def _tc_call(body, out_shape, *args):
    return pl.pallas_call(body, out_shape=out_shape)(*args)


def kernel(x, ei, batch, W1, b1, W2, b2, Wl, bl):
    pad = EPAD - E
    idx = jnp.arange(pad, dtype=jnp.int32)
    src_flat = jnp.concatenate([ei[0], idx])
    dst_flat = jnp.concatenate([ei[1], N + (idx & 127)])
    src_p = src_flat.reshape(NW, NCHUNK, CH)
    dst_p = dst_flat.reshape(NW, NCHUNK, CH)
    dst_deg = ei[1].reshape(NW, NCHD, CHD)
    x_p = jnp.pad(x, ((0, NPAD - N), (0, 0)))
    batch_p = jnp.pad(batch, (0, NPAD - N), constant_values=B).reshape(
        NPAD, 1)

    degp = _deg_call(dst_deg)
    dinv = _tc_call(_tcdeg_body,
                    jax.ShapeDtypeStruct((CHD, D), jnp.float32),
                    degp).reshape(NPAD, 1)

    h1 = _tc_call(_tc0_body, jax.ShapeDtypeStruct((NPAD, D), jnp.float32),
                  x_p, W1)
    g1 = _tc_call(_tc1_body, jax.ShapeDtypeStruct((NPAD, D), jnp.float32),
                  h1, dinv)
    p1 = _agg_call(g1, src_p, dst_p)
    g2 = _tc_call(_tc2_body, jax.ShapeDtypeStruct((NPAD, D), jnp.float32),
                  dinv, p1, b1.reshape(1, D), W2)
    p2 = _agg_call(g2, src_p, dst_p)
    out = _tc_call(_tc3_body, jax.ShapeDtypeStruct((B, D), jnp.float32),
                   dinv, p2, b2.reshape(1, D), batch_p, Wl,
                   bl.reshape(1, D))
    return out
